# Initial kernel scaffold; baseline (speedup 1.0000x reference)
import jax
import jax.numpy as jnp
from jax.experimental import pallas as pl


def kernel(coordinates, gene_ix, weight1):
    raise NotImplementedError("write your pallas kernel here")


# speedup vs baseline: 11.9202x; 11.9202x over previous
import functools

import numpy as np
import jax
import jax.numpy as jnp
from jax import lax
from jax.experimental import pallas as pl
from jax.experimental.pallas import tpu as pltpu
from jax.experimental.pallas import tpu_sc as plsc

N_GENES = 100000
N_FREQ = 10
D_IN = 4 * N_FREQ
D_EMB = 16
N_FRAG = 200000

NC = 2
NS = 16
NW = NC * NS
N_PAD = 200704
B_W = N_PAD // NW
CH = 64
N_CH = B_W // CH


def _freq_shift_consts():
    freqs = np.array(
        [[1.0 / 1000.0 ** (2.0 * i / N_FREQ)] * 2 for i in range(1, N_FREQ + 1)],
        dtype=np.float32).reshape(-1)
    shifts = np.array(
        [[0.0, np.pi / 2.0] for _ in range(1, N_FREQ + 1)],
        dtype=np.float32).reshape(-1)
    return freqs, shifts


_ENC_BLK = 2048


def _enc_body(coord_ref, out_ref):
    j = lax.broadcasted_iota(jnp.int32, (1, 2 * N_FREQ), 1)
    half = j // 2
    i = (half + 1).astype(jnp.float32)
    f = jnp.exp(i * jnp.float32(-2.0 / N_FREQ * np.log(1000.0)))
    s = (j - 2 * half).astype(jnp.float32) * jnp.float32(np.pi / 2.0)
    c = coord_ref[...]
    e0 = jnp.sin(c[:, 0:1] * f + s)
    e1 = jnp.sin(c[:, 1:2] * f + s)
    out_ref[...] = jnp.concatenate([e0, e1], axis=1)


def _encode(coords_p):
    return pl.pallas_call(
        _enc_body,
        grid=(N_PAD // _ENC_BLK,),
        in_specs=[pl.BlockSpec((_ENC_BLK, 2), lambda i: (i, 0))],
        out_specs=pl.BlockSpec((_ENC_BLK, D_IN), lambda i: (i, 0)),
        out_shape=jax.ShapeDtypeStruct((N_PAD, D_IN), jnp.float32),
    )(coords_p)


_SC_MESH = plsc.VectorSubcoreMesh(
    core_axis_name="c", subcore_axis_name="s", num_cores=NC, num_subcores=NS)


@functools.partial(
    pl.kernel,
    out_type=jax.ShapeDtypeStruct((N_PAD, D_EMB), jnp.float32),
    mesh=_SC_MESH,
    scratch_types=[
        pltpu.VMEM((CH,), jnp.int32),
        pltpu.VMEM((CH, D_IN * D_EMB), jnp.float32),
        pltpu.VMEM((CH, D_IN), jnp.float32),
        pltpu.VMEM((CH, D_EMB), jnp.float32),
        pltpu.SemaphoreType.DMA,
    ],
)
def _sc_embed(table_hbm, idx_hbm, emb_hbm, out_hbm,
              idxc_v, rows_v, emb_v, outb_v, sem):
    wid = lax.axis_index("s") * NC + lax.axis_index("c")
    base = wid * B_W

    def chunk_body(c, carry):
        off = base + c * CH
        pltpu.sync_copy(idx_hbm.at[pl.ds(off, CH)], idxc_v)
        gather = pltpu.async_copy(table_hbm.at[idxc_v], rows_v, sem)
        pltpu.sync_copy(emb_hbm.at[pl.ds(off, CH)], emb_v)
        gather.wait()

        def frag_body(j, carry2):
            ev0 = emb_v[j, pl.ds(0, 16)]
            ev1 = emb_v[j, pl.ds(16, 16)]
            ev2 = emb_v[j, pl.ds(24, 16)]
            acc = jnp.zeros((D_EMB,), jnp.float32)
            for k in range(D_IN):
                if k < 16:
                    e = ev0[k]
                elif k < 32:
                    e = ev1[k - 16]
                else:
                    e = ev2[k - 24]
                w = rows_v[j, pl.ds(k * D_EMB, D_EMB)]
                acc = acc + e * w
            outb_v[j, :] = jnp.maximum(acc, 0.0)
            return carry2

        lax.fori_loop(0, CH, frag_body, 0, unroll=False)
        pltpu.sync_copy(outb_v, out_hbm.at[pl.ds(off, CH)])
        return carry

    lax.fori_loop(0, N_CH, chunk_body, 0, unroll=False)


def kernel(coordinates, gene_ix, weight1):
    coords_p = jnp.pad(coordinates, ((0, N_PAD - N_FRAG), (0, 0)))
    gene_p = jnp.pad(gene_ix, (0, N_PAD - N_FRAG))
    emb = _encode(coords_p)
    table = weight1.reshape(N_GENES, D_IN * D_EMB)
    out = _sc_embed(table, gene_p, emb)
    return out[:N_FRAG]


# speedup vs baseline: 13.1910x; 1.1066x over previous
import functools

import numpy as np
import jax
import jax.numpy as jnp
from jax import lax
from jax.experimental import pallas as pl
from jax.experimental.pallas import tpu as pltpu
from jax.experimental.pallas import tpu_sc as plsc

N_GENES = 100000
N_FREQ = 10
D_IN = 4 * N_FREQ
D_EMB = 16
N_FRAG = 200000

NC = 2
NS = 16
NW = NC * NS
N_PAD = 204800
B_W = N_PAD // NW
CH = 64
N_CH = B_W // CH
NPAIR = N_CH // 2


_ENC_BLK = 2048


def _enc_body(coord_ref, out_ref):
    j = lax.broadcasted_iota(jnp.int32, (1, 2 * N_FREQ), 1)
    half = j // 2
    i = (half + 1).astype(jnp.float32)
    f = jnp.exp(i * jnp.float32(-2.0 / N_FREQ * np.log(1000.0)))
    s = (j - 2 * half).astype(jnp.float32) * jnp.float32(np.pi / 2.0)
    c = coord_ref[...]
    e0 = jnp.sin(c[:, 0:1] * f + s)
    e1 = jnp.sin(c[:, 1:2] * f + s)
    out_ref[...] = jnp.concatenate([e0, e1], axis=1)


def _encode(coords_p):
    return pl.pallas_call(
        _enc_body,
        grid=(N_PAD // _ENC_BLK,),
        in_specs=[pl.BlockSpec((_ENC_BLK, 2), lambda i: (i, 0))],
        out_specs=pl.BlockSpec((_ENC_BLK, D_IN), lambda i: (i, 0)),
        out_shape=jax.ShapeDtypeStruct((N_PAD, D_IN), jnp.float32),
    )(coords_p)


_SC_MESH = plsc.VectorSubcoreMesh(
    core_axis_name="c", subcore_axis_name="s", num_cores=NC, num_subcores=NS)


@functools.partial(
    pl.kernel,
    out_type=jax.ShapeDtypeStruct((N_PAD, D_EMB), jnp.float32),
    mesh=_SC_MESH,
    scratch_types=[
        pltpu.VMEM((B_W,), jnp.int32),
        pltpu.VMEM((CH, D_IN * D_EMB), jnp.float32),
        pltpu.VMEM((CH, D_IN * D_EMB), jnp.float32),
        pltpu.VMEM((CH, D_IN), jnp.float32),
        pltpu.VMEM((CH, D_IN), jnp.float32),
        pltpu.VMEM((CH, D_EMB), jnp.float32),
        pltpu.VMEM((CH, D_EMB), jnp.float32),
        pltpu.SemaphoreType.DMA,
        pltpu.SemaphoreType.DMA,
        pltpu.SemaphoreType.DMA,
        pltpu.SemaphoreType.DMA,
        pltpu.SemaphoreType.DMA,
        pltpu.SemaphoreType.DMA,
    ],
)
def _sc_embed(table_hbm, idx_hbm, emb_hbm, out_hbm,
              idx_v, rows0, rows1, emb0, emb1, outb0, outb1,
              sg0, sg1, se0, se1, so0, so1):
    wid = lax.axis_index("s") * NC + lax.axis_index("c")
    base = wid * B_W
    pltpu.sync_copy(idx_hbm.at[pl.ds(base, B_W)], idx_v)

    def issue(c, rows_b, emb_b, sg, se):
        pltpu.async_copy(table_hbm.at[idx_v.at[pl.ds(c * CH, CH)]], rows_b, sg)
        pltpu.async_copy(emb_hbm.at[pl.ds(base + c * CH, CH)], emb_b, se)

    def compute(rows_b, emb_b, out_b):
        def frag_body(j, carry2):
            ev0 = emb_b[j, pl.ds(0, 16)]
            ev1 = emb_b[j, pl.ds(16, 16)]
            ev2 = emb_b[j, pl.ds(24, 16)]
            accs = [None, None, None, None]
            for k in range(D_IN):
                if k < 16:
                    e = ev0[k]
                elif k < 32:
                    e = ev1[k - 16]
                else:
                    e = ev2[k - 24]
                t = e * rows_b[j, pl.ds(k * D_EMB, D_EMB)]
                a = k % 4
                accs[a] = t if accs[a] is None else accs[a] + t
            acc = (accs[0] + accs[1]) + (accs[2] + accs[3])
            out_b[j, :] = jnp.maximum(acc, 0.0)
            return carry2

        lax.fori_loop(0, CH, frag_body, 0)

    def process(c, rows_b, emb_b, out_b, sg, se, so, have_prev_store):
        pltpu.make_async_copy(
            table_hbm.at[idx_v.at[pl.ds(c * CH, CH)]], rows_b, sg).wait()
        pltpu.make_async_copy(
            emb_hbm.at[pl.ds(base + c * CH, CH)], emb_b, se).wait()

        @pl.when(have_prev_store)
        def _():
            pltpu.make_async_copy(
                out_b, out_hbm.at[pl.ds(base + c * CH, CH)], so).wait()

        compute(rows_b, emb_b, out_b)
        pltpu.async_copy(out_b, out_hbm.at[pl.ds(base + c * CH, CH)], so)

    issue(0, rows0, emb0, sg0, se0)

    def pair_body(p, carry):
        c0 = 2 * p
        issue(c0 + 1, rows1, emb1, sg1, se1)
        process(c0, rows0, emb0, outb0, sg0, se0, so0, p > 0)

        @pl.when(p < NPAIR - 1)
        def _():
            issue(c0 + 2, rows0, emb0, sg0, se0)

        process(c0 + 1, rows1, emb1, outb1, sg1, se1, so1, p > 0)
        return carry

    lax.fori_loop(0, NPAIR, pair_body, 0)
    pltpu.make_async_copy(
        outb0, out_hbm.at[pl.ds(base + (N_CH - 2) * CH, CH)], so0).wait()
    pltpu.make_async_copy(
        outb1, out_hbm.at[pl.ds(base + (N_CH - 1) * CH, CH)], so1).wait()


def kernel(coordinates, gene_ix, weight1):
    coords_p = jnp.pad(coordinates, ((0, N_PAD - N_FRAG), (0, 0)))
    gene_p = jnp.pad(gene_ix, (0, N_PAD - N_FRAG))
    emb = _encode(coords_p)
    table = weight1.reshape(N_GENES, D_IN * D_EMB)
    out = _sc_embed(table, gene_p, emb)
    return out[:N_FRAG]
